# baseline (speedup 1.0000x reference)
import functools
import math

import jax
import jax.numpy as jnp
from jax import lax
from jax.experimental import pallas as pl
from jax.experimental.pallas import tpu as pltpu
from jax.experimental.pallas import tpu_sc as plsc

N = 10000
NP = 10240
RROWS = NP // 128
H = 128
G = 16
E = 320000
NW = 32
EPW = 10112
EP = NW * EPW
NCH = EPW // 128
EPS = 1e-5
F32 = jnp.float32
I32 = jnp.int32
NCH_C0 = 52
NCH_C1 = 106

_SC_MESH = dict(core_axis_name="c", subcore_axis_name="s", num_cores=2,
                num_subcores=16)
_SC_PARAMS = pltpu.CompilerParams(needs_layout_passes=False)


@functools.partial(
    pl.kernel,
    out_type=(jax.ShapeDtypeStruct((EP,), F32),
              jax.ShapeDtypeStruct((2 * NP,), F32)),
    mesh=plsc.VectorSubcoreMesh(**_SC_MESH),
    scratch_types=[
        pltpu.VMEM((NP,), F32),
        pltpu.VMEM((NP,), F32),
        pltpu.VMEM((EPW,), I32),
        pltpu.VMEM((EPW,), I32),
        pltpu.VMEM((EPW,), F32),
        pltpu.VMEM((EPW,), F32),
        pltpu.VMEM((640,), F32),
        pltpu.VMEM((640,), F32),
        pltpu.VMEM_SHARED((16, NP), F32),
    ],
    compiler_params=_SC_PARAMS,
)
def _sc_deg(row_hbm, col_hbm, ew_hbm, kept_hbm, ewn_hbm, deg_hbm,
            kept_v, deg_v, row_v, col_v, ew_v, ewn_v, t_v, a_v, shared):
    cid = lax.axis_index("c")
    sid = lax.axis_index("s")
    wid = sid * 2 + cid
    base = wid * EPW
    pltpu.sync_copy(kept_hbm, kept_v)
    pltpu.sync_copy(row_hbm.at[pl.ds(base, EPW)], row_v)
    pltpu.sync_copy(col_hbm.at[pl.ds(base, EPW)], col_v)
    pltpu.sync_copy(ew_hbm.at[pl.ds(base, EPW)], ew_v)

    def zero_deg(i, _):
        deg_v[pl.ds(i * 16, 16)] = jnp.zeros((16,), F32)
        return 0
    lax.fori_loop(0, NP // 16, zero_deg, 0)

    def edge_body(i, _):
        sl = pl.ds(i * 16, 16)
        r = row_v[sl]
        c = col_v[sl]
        w = ew_v[sl]
        kr = plsc.load_gather(kept_v, [r])
        kc = plsc.load_gather(kept_v, [c])
        wn = jnp.where(r == c, jnp.float32(0.0), w) * kr * kc
        ewn_v[sl] = wn
        plsc.addupdate_scatter(deg_v, [c], wn)
        return 0
    lax.fori_loop(0, EPW // 16, edge_body, 0)

    pltpu.sync_copy(ewn_v, ewn_hbm.at[pl.ds(base, EPW)])
    pltpu.sync_copy(deg_v, shared.at[sid])
    plsc.subcore_barrier()

    def zero_a(i, _):
        a_v[pl.ds(i * 16, 16)] = jnp.zeros((16,), F32)
        return 0
    lax.fori_loop(0, 40, zero_a, 0)
    for t in range(16):
        pltpu.sync_copy(shared.at[t, pl.ds(sid * 640, 640)], t_v)

        def add_t(j, _):
            sl = pl.ds(j * 16, 16)
            a_v[sl] = a_v[sl] + t_v[sl]
            return 0
        lax.fori_loop(0, 40, add_t, 0)
    pltpu.sync_copy(a_v, deg_hbm.at[pl.ds(cid * NP + sid * 640, 640)])


@functools.partial(
    pl.kernel,
    out_type=jax.ShapeDtypeStruct((2, NP, H), F32),
    mesh=plsc.VectorSubcoreMesh(**_SC_MESH),
    scratch_types=[
        pltpu.VMEM((128,), I32),
        pltpu.VMEM((128,), I32),
        pltpu.VMEM((128,), I32),
        pltpu.VMEM((128,), I32),
        pltpu.VMEM((128,), F32),
        pltpu.VMEM((128,), F32),
        pltpu.VMEM((128, H), F32),
        pltpu.VMEM((128, H), F32),
        pltpu.VMEM_SHARED((NP, H), F32),
        pltpu.SemaphoreType.DMA,
        pltpu.SemaphoreType.DMA,
        pltpu.SemaphoreType.DMA,
        pltpu.SemaphoreType.DMA,
        pltpu.SemaphoreType.DMA,
        pltpu.SemaphoreType.DMA,
    ],
    compiler_params=_SC_PARAMS,
)
def _sc_agg(y_hbm, row_hbm, col_hbm, ew_hbm, agg_hbm,
            ridx0_v, ridx1_v, cidx0_v, cidx1_v, ew0_v, ew1_v,
            rows0_v, rows1_v, acc, isem0, isem1, gsem0, gsem1, ssem0, ssem1):
    cid = lax.axis_index("c")
    sid = lax.axis_index("s")
    nch = jnp.where(cid == 0, NCH_C0, NCH_C1)
    chunk0 = jnp.where(cid == 0, sid * NCH_C0,
                       16 * NCH_C0 + sid * NCH_C1)

    def zero_rows(i, _):
        for j in range(8):
            rows0_v[i, pl.ds(j * 16, 16)] = jnp.zeros((16,), F32)
        return 0
    lax.fori_loop(0, 128, zero_rows, 0)
    for t in range(5):
        pltpu.sync_copy(rows0_v, acc.at[pl.ds(sid * 640 + t * 128, 128), :])
    plsc.subcore_barrier()

    ridx = (ridx0_v, ridx1_v)
    cidx = (cidx0_v, cidx1_v)
    ewch = (ew0_v, ew1_v)
    rows = (rows0_v, rows1_v)
    isem = (isem0, isem1)
    gsem = (gsem0, gsem1)
    ssem = (ssem0, ssem1)

    def stage(m, b):
        off = (chunk0 + m) * 128
        pltpu.async_copy(row_hbm.at[pl.ds(off, 128)], ridx[b], isem[b])
        pltpu.async_copy(col_hbm.at[pl.ds(off, 128)], cidx[b], isem[b])
        pltpu.async_copy(ew_hbm.at[pl.ds(off, 128)], ewch[b], isem[b])
        pltpu.make_async_copy(row_hbm.at[pl.ds(off, 128)], ridx[b],
                              isem[b]).wait()
        pltpu.make_async_copy(col_hbm.at[pl.ds(off, 128)], cidx[b],
                              isem[b]).wait()
        pltpu.make_async_copy(ew_hbm.at[pl.ds(off, 128)], ewch[b],
                              isem[b]).wait()
        pltpu.async_copy(y_hbm.at[ridx[b]], rows[b], gsem[b])

    def scale(m, b):
        def escale(t, _):
            eb = t * 16
            w16 = ewch[b][pl.ds(eb, 16)]
            for l in range(16):
                w = w16[l]
                for j in range(8):
                    sl = pl.ds(j * 16, 16)
                    rows[b][eb + l, sl] = rows[b][eb + l, sl] * w
            return 0
        lax.fori_loop(0, 8, escale, 0)

    def gwait(b):
        pltpu.make_async_copy(y_hbm.at[ridx[b]], rows[b], gsem[b]).wait()

    def swait(b):
        pltpu.make_async_copy(rows[b], acc.at[cidx[b]], ssem[b]).wait()

    stage(0, 0)

    def pair(i, _):
        m0 = i * 2

        @pl.when(i > 0)
        def _():
            swait(1)
        stage(m0 + 1, 1)
        gwait(0)
        scale(m0, 0)
        pltpu.async_copy(rows[0], acc.at[cidx[0]], ssem[0], add=True)

        @pl.when(m0 + 2 < nch)
        def _():
            swait(0)
            stage(m0 + 2, 0)
        gwait(1)
        scale(m0 + 1, 1)
        pltpu.async_copy(rows[1], acc.at[cidx[1]], ssem[1], add=True)
        return 0
    lax.fori_loop(0, nch // 2, pair, 0)
    swait(0)
    swait(1)
    plsc.subcore_barrier()
    pltpu.sync_copy(acc.at[pl.ds(sid * 640, 640), :],
                    agg_hbm.at[cid].at[pl.ds(sid * 640, 640), :])


_TC_PARAMS = pltpu.CompilerParams(vmem_limit_bytes=100 * 1024 * 1024)


def _tc0_body(batch_ref, counts_ref):
    bm = batch_ref[...]
    for g in range(G):
        counts_ref[g] = jnp.sum((bm == g).astype(I32))


def _counts_from_batch(batch_m):
    return pl.pallas_call(
        _tc0_body,
        out_shape=jax.ShapeDtypeStruct((G,), I32),
        in_specs=[pl.BlockSpec(memory_space=pltpu.VMEM)],
        out_specs=pl.BlockSpec(memory_space=pltpu.SMEM),
        compiler_params=_TC_PARAMS,
    )(batch_m)


def _tcpre_body(h_ref, w_ref, deg_ref, y_ref, dinv_ref):
    xw = jnp.dot(h_ref[...], w_ref[...], preferred_element_type=F32)
    deg = deg_ref[0] + deg_ref[1] + 1.0
    dinv = lax.rsqrt(deg)
    dinv_ref[...] = dinv
    y3 = xw.reshape(RROWS, 128, H) * dinv[:, :, None]
    y_ref[...] = y3.reshape(NP, H)


def _tc_pre(h, Wc, deg3):
    return pl.pallas_call(
        _tcpre_body,
        out_shape=(jax.ShapeDtypeStruct((NP, H), F32),
                   jax.ShapeDtypeStruct((RROWS, 128), F32)),
        in_specs=[pl.BlockSpec(memory_space=pltpu.VMEM)] * 3,
        out_specs=(pl.BlockSpec(memory_space=pltpu.VMEM),) * 2,
        compiler_params=_TC_PARAMS,
    )(h, Wc, deg3)


def _tc3_body(agg0_ref, agg1_ref, y_ref, dinv_ref, bc_ref, gm_ref, bt_ref,
              p_ref, alive_ref, batch_ref, counts_ref, pn_ref, flat_ref,
              h_ref, kept_ref, cnew_ref, fout_ref, s_scr, bb_scr):
    dinv = dinv_ref[...]
    conv = dinv[:, :, None] * (agg0_ref[...] + agg1_ref[...] + y_ref[...])
    conv = conv + bc_ref[...].reshape(1, 1, H)
    alive = alive_ref[...]
    alive3 = alive[:, :, None]
    cnt = counts_ref[0]
    for g in range(1, G):
        cnt = cnt + counts_ref[g]
    cntf = jnp.maximum(cnt.astype(F32), 1.0)
    mean = jnp.sum(jnp.sum(conv * alive3, axis=0), axis=0) / cntf
    cen = conv - mean[None, None, :]
    var = jnp.sum(jnp.sum(cen * cen * alive3, axis=0), axis=0) / cntf
    gm = gm_ref[...].reshape(1, 1, H)
    bt = bt_ref[...].reshape(1, 1, H)
    h2 = jax.nn.relu(gm * cen / jnp.sqrt(var + EPS)[None, None, :] + bt)

    p2 = p_ref[...]
    zrows = []
    for r in range(RROWS):
        zrows.append(lax.dot_general(p2, h2[r], (((1,), (1,)), ((), ())),
                                     preferred_element_type=F32))
    z = jnp.concatenate(zrows, axis=0)
    s = jnp.tanh(z / pn_ref[0, 0])
    s_scr[...] = s

    bm = batch_ref[...]
    bb = jnp.where(alive > 0.0, bm, -1)
    bb_scr[...] = bb
    iidx = (lax.broadcasted_iota(I32, (RROWS, 128), 0) * 128
            + lax.broadcasted_iota(I32, (RROWS, 128), 1))
    s3 = s[:, :, None]
    i3 = iidx[:, :, None]
    b3 = bb[:, :, None]

    def jbody(jt, rank):
        sj = s_scr[pl.ds(jt, 1), :].reshape(1, 1, 128)
        bj = bb_scr[pl.ds(jt, 1), :].reshape(1, 1, 128)
        ij = (jt * 128
              + lax.broadcasted_iota(I32, (1, 1, 128), 2))
        above = (sj > s3) | ((sj == s3) & (ij < i3))
        return rank + jnp.sum((above & (bj == b3)).astype(I32), axis=2)

    rank = lax.fori_loop(0, RROWS, jbody, jnp.zeros((RROWS, 128), I32))

    kvals = []
    for g in range(G):
        kg = (4 * counts_ref[g] + 4) // 5
        kvals.append(kg)
        cnew_ref[g] = kg
    kb = jnp.zeros((RROWS, 128), I32)
    for g in range(G):
        kb = jnp.where(bm == g, kvals[g], kb)
    kept = (alive > 0.0) & (rank < kb)
    keptf = kept.astype(F32)
    kept_ref[...] = keptf
    h_new = h2 * s3 * keptf[:, :, None]
    h_ref[...] = h_new.reshape(NP, H)

    means = []
    maxs = []
    ninf = jnp.float32(-jnp.inf)
    for g in range(G):
        mf3 = (kept & (bm == g)).astype(F32)[:, :, None]
        sum_g = jnp.sum(jnp.sum(h_new * mf3, axis=0), axis=0)
        kf = jnp.maximum(kvals[g].astype(F32), 1.0)
        means.append(sum_g / kf)
        maxs.append(jnp.max(jnp.max(jnp.where(mf3 > 0.5, h_new, ninf), axis=0),
                            axis=0))
    flat = jnp.concatenate([jnp.stack(means), jnp.stack(maxs)], axis=1)
    fout_ref[...] = flat_ref[...] + flat


def _tc_post(agg0, agg1, y3, dinv_m, bc, gm, bt, p, alive_m, batch_m, counts,
             pn, flat):
    return pl.pallas_call(
        _tc3_body,
        out_shape=(jax.ShapeDtypeStruct((NP, H), F32),
                   jax.ShapeDtypeStruct((RROWS, 128), F32),
                   jax.ShapeDtypeStruct((G,), I32),
                   jax.ShapeDtypeStruct((G, 2 * H), F32)),
        in_specs=([pl.BlockSpec(memory_space=pltpu.VMEM)] * 10
                  + [pl.BlockSpec(memory_space=pltpu.SMEM),
                     pl.BlockSpec(memory_space=pltpu.SMEM),
                     pl.BlockSpec(memory_space=pltpu.VMEM)]),
        out_specs=(pl.BlockSpec(memory_space=pltpu.VMEM),
                   pl.BlockSpec(memory_space=pltpu.VMEM),
                   pl.BlockSpec(memory_space=pltpu.SMEM),
                   pl.BlockSpec(memory_space=pltpu.VMEM)),
        scratch_shapes=[pltpu.VMEM((RROWS, 128), F32),
                        pltpu.VMEM((RROWS, 128), I32)],
        compiler_params=_TC_PARAMS,
    )(agg0, agg1, y3, dinv_m, bc, gm, bt, p, alive_m, batch_m, counts, pn,
      flat)


def _mlp_body(f_ref, w1_ref, b1_ref, w2_ref, b2_ref, o_ref):
    hid = jax.nn.relu(
        jnp.dot(f_ref[...], w1_ref[...], preferred_element_type=F32)
        + b1_ref[...])
    o_ref[...] = (jnp.dot(hid, w2_ref[...], preferred_element_type=F32)
                  + b2_ref[...])


def _mlp(flat, W1, b1, W2, b2):
    return pl.pallas_call(
        _mlp_body,
        out_shape=jax.ShapeDtypeStruct((G, W2.shape[1]), F32),
        in_specs=[pl.BlockSpec(memory_space=pltpu.VMEM)] * 5,
        out_specs=pl.BlockSpec(memory_space=pltpu.VMEM),
        compiler_params=_TC_PARAMS,
    )(flat, W1, b1[None, :], W2, b2[None, :])


def kernel(inputs, edge_index, batch, edge_weight, Wc0, bc0, gamma0, beta0,
           p0, Wc1, bc1, gamma1, beta1, p1, Wc2, bc2, gamma2, beta2, p2,
           W1, b1, W2, b2):
    row = edge_index[0].astype(I32)
    col = edge_index[1].astype(I32)
    zpad = jnp.zeros((EP - E,), I32)
    row_p = jnp.concatenate([row, zpad])
    col_p = jnp.concatenate([col, zpad])
    ew_p = jnp.concatenate([edge_weight.astype(F32),
                            jnp.zeros((EP - E,), F32)])
    batch_m = jnp.concatenate([batch.astype(I32),
                               jnp.full((NP - N,), G, I32)]).reshape(RROWS, 128)
    h = jnp.concatenate([inputs.astype(F32), jnp.zeros((NP - N, H), F32)],
                        axis=0)
    alive_m = (jnp.arange(NP) < N).astype(F32).reshape(RROWS, 128)

    counts = _counts_from_batch(batch_m)
    flat = jnp.zeros((G, 2 * H), F32)
    ew_cur = ew_p
    blocks = [(Wc0, bc0, gamma0, beta0, p0), (Wc1, bc1, gamma1, beta1, p1),
              (Wc2, bc2, gamma2, beta2, p2)]
    for (Wc, bc, gm, bt, p) in blocks:
        ewn, degflat = _sc_deg(row_p, col_p, ew_cur, alive_m.reshape(NP))
        y, dinv_m = _tc_pre(h, Wc, degflat.reshape(2, RROWS, 128))
        aggpart = _sc_agg(y, row_p, col_p, ewn)
        pn = jnp.linalg.norm(p).reshape(1, 1)
        h, alive_m, counts, flat = _tc_post(
            aggpart[0].reshape(RROWS, 128, H), aggpart[1].reshape(RROWS, 128, H),
            y.reshape(RROWS, 128, H), dinv_m, bc.reshape(1, H),
            gm.reshape(1, H), bt.reshape(1, H), p.reshape(1, H),
            alive_m, batch_m, counts, pn, flat)
        ew_cur = ewn
    return _mlp(flat, W1, b1, W2, b2)


# speedup vs baseline: 1.1446x; 1.1446x over previous
import functools
import math

import jax
import jax.numpy as jnp
from jax import lax
from jax.experimental import pallas as pl
from jax.experimental.pallas import tpu as pltpu
from jax.experimental.pallas import tpu_sc as plsc

N = 10000
NP = 10240
RROWS = NP // 128
H = 128
G = 16
E = 320000
NW = 32
EPW = 10112
EP = NW * EPW
NCH = EPW // 128
EPS = 1e-5
F32 = jnp.float32
I32 = jnp.int32
NCH_C0 = 106
NCH_C1 = 52

_SC_MESH = dict(core_axis_name="c", subcore_axis_name="s", num_cores=2,
                num_subcores=16)
_SC_PARAMS = pltpu.CompilerParams(needs_layout_passes=False)


@functools.partial(
    pl.kernel,
    out_type=(jax.ShapeDtypeStruct((EP,), F32),
              jax.ShapeDtypeStruct((2 * NP,), F32)),
    mesh=plsc.VectorSubcoreMesh(**_SC_MESH),
    scratch_types=[
        pltpu.VMEM((NP,), F32),
        pltpu.VMEM((NP,), F32),
        pltpu.VMEM((EPW,), I32),
        pltpu.VMEM((EPW,), I32),
        pltpu.VMEM((EPW,), F32),
        pltpu.VMEM((EPW,), F32),
        pltpu.VMEM((640,), F32),
        pltpu.VMEM((640,), F32),
        pltpu.VMEM_SHARED((16, NP), F32),
    ],
    compiler_params=_SC_PARAMS,
)
def _sc_deg(row_hbm, col_hbm, ew_hbm, kept_hbm, ewn_hbm, deg_hbm,
            kept_v, deg_v, row_v, col_v, ew_v, ewn_v, t_v, a_v, shared):
    cid = lax.axis_index("c")
    sid = lax.axis_index("s")
    wid = sid * 2 + cid
    base = wid * EPW
    pltpu.sync_copy(kept_hbm, kept_v)
    pltpu.sync_copy(row_hbm.at[pl.ds(base, EPW)], row_v)
    pltpu.sync_copy(col_hbm.at[pl.ds(base, EPW)], col_v)
    pltpu.sync_copy(ew_hbm.at[pl.ds(base, EPW)], ew_v)

    def zero_deg(i, _):
        deg_v[pl.ds(i * 16, 16)] = jnp.zeros((16,), F32)
        return 0
    lax.fori_loop(0, NP // 16, zero_deg, 0)

    def edge_body(i, _):
        sl = pl.ds(i * 16, 16)
        r = row_v[sl]
        c = col_v[sl]
        w = ew_v[sl]
        kr = plsc.load_gather(kept_v, [r])
        kc = plsc.load_gather(kept_v, [c])
        wn = jnp.where(r == c, jnp.float32(0.0), w) * kr * kc
        ewn_v[sl] = wn
        plsc.addupdate_scatter(deg_v, [c], wn)
        return 0
    lax.fori_loop(0, EPW // 16, edge_body, 0)

    pltpu.sync_copy(ewn_v, ewn_hbm.at[pl.ds(base, EPW)])
    pltpu.sync_copy(deg_v, shared.at[sid])
    plsc.subcore_barrier()

    def zero_a(i, _):
        a_v[pl.ds(i * 16, 16)] = jnp.zeros((16,), F32)
        return 0
    lax.fori_loop(0, 40, zero_a, 0)
    for t in range(16):
        pltpu.sync_copy(shared.at[t, pl.ds(sid * 640, 640)], t_v)

        def add_t(j, _):
            sl = pl.ds(j * 16, 16)
            a_v[sl] = a_v[sl] + t_v[sl]
            return 0
        lax.fori_loop(0, 40, add_t, 0)
    pltpu.sync_copy(a_v, deg_hbm.at[pl.ds(cid * NP + sid * 640, 640)])


@functools.partial(
    pl.kernel,
    out_type=jax.ShapeDtypeStruct((2, NP, H), F32),
    mesh=plsc.VectorSubcoreMesh(**_SC_MESH),
    scratch_types=[
        pltpu.VMEM((128,), I32),
        pltpu.VMEM((128,), I32),
        pltpu.VMEM((128,), I32),
        pltpu.VMEM((128,), I32),
        pltpu.VMEM((128,), F32),
        pltpu.VMEM((128,), F32),
        pltpu.VMEM((128, H), F32),
        pltpu.VMEM((128, H), F32),
        pltpu.VMEM_SHARED((NP, H), F32),
        pltpu.SemaphoreType.DMA,
        pltpu.SemaphoreType.DMA,
        pltpu.SemaphoreType.DMA,
        pltpu.SemaphoreType.DMA,
        pltpu.SemaphoreType.DMA,
        pltpu.SemaphoreType.DMA,
    ],
    compiler_params=_SC_PARAMS,
)
def _sc_agg(y_hbm, row_hbm, col_hbm, ew_hbm, agg_hbm,
            ridx0_v, ridx1_v, cidx0_v, cidx1_v, ew0_v, ew1_v,
            rows0_v, rows1_v, acc, isem0, isem1, gsem0, gsem1, ssem0, ssem1):
    cid = lax.axis_index("c")
    sid = lax.axis_index("s")
    nch = jnp.where(cid == 0, NCH_C0, NCH_C1)
    chunk0 = jnp.where(cid == 0, sid * NCH_C0,
                       16 * NCH_C0 + sid * NCH_C1)

    def zero_rows(i, _):
        for j in range(8):
            rows0_v[i, pl.ds(j * 16, 16)] = jnp.zeros((16,), F32)
        return 0
    lax.fori_loop(0, 128, zero_rows, 0)
    for t in range(5):
        pltpu.sync_copy(rows0_v, acc.at[pl.ds(sid * 640 + t * 128, 128), :])
    plsc.subcore_barrier()

    ridx = (ridx0_v, ridx1_v)
    cidx = (cidx0_v, cidx1_v)
    ewch = (ew0_v, ew1_v)
    rows = (rows0_v, rows1_v)
    isem = (isem0, isem1)
    gsem = (gsem0, gsem1)
    ssem = (ssem0, ssem1)

    def stage(m, b):
        off = (chunk0 + m) * 128
        pltpu.async_copy(row_hbm.at[pl.ds(off, 128)], ridx[b], isem[b])
        pltpu.async_copy(col_hbm.at[pl.ds(off, 128)], cidx[b], isem[b])
        pltpu.async_copy(ew_hbm.at[pl.ds(off, 128)], ewch[b], isem[b])
        pltpu.make_async_copy(row_hbm.at[pl.ds(off, 128)], ridx[b],
                              isem[b]).wait()
        pltpu.make_async_copy(col_hbm.at[pl.ds(off, 128)], cidx[b],
                              isem[b]).wait()
        pltpu.make_async_copy(ew_hbm.at[pl.ds(off, 128)], ewch[b],
                              isem[b]).wait()
        pltpu.async_copy(y_hbm.at[ridx[b]], rows[b], gsem[b])

    def scale(m, b):
        def escale(t, _):
            eb = t * 16
            w16 = ewch[b][pl.ds(eb, 16)]
            for l in range(16):
                w = w16[l]
                for j in range(8):
                    sl = pl.ds(j * 16, 16)
                    rows[b][eb + l, sl] = rows[b][eb + l, sl] * w
            return 0
        lax.fori_loop(0, 8, escale, 0)

    def gwait(b):
        pltpu.make_async_copy(y_hbm.at[ridx[b]], rows[b], gsem[b]).wait()

    def swait(b):
        pltpu.make_async_copy(rows[b], acc.at[cidx[b]], ssem[b]).wait()

    stage(0, 0)

    def pair(i, _):
        m0 = i * 2

        @pl.when(i > 0)
        def _():
            swait(1)
        stage(m0 + 1, 1)
        gwait(0)
        scale(m0, 0)
        pltpu.async_copy(rows[0], acc.at[cidx[0]], ssem[0], add=True)

        @pl.when(m0 + 2 < nch)
        def _():
            swait(0)
            stage(m0 + 2, 0)
        gwait(1)
        scale(m0 + 1, 1)
        pltpu.async_copy(rows[1], acc.at[cidx[1]], ssem[1], add=True)
        return 0
    lax.fori_loop(0, nch // 2, pair, 0)
    swait(0)
    swait(1)
    plsc.subcore_barrier()
    pltpu.sync_copy(acc.at[pl.ds(sid * 640, 640), :],
                    agg_hbm.at[cid].at[pl.ds(sid * 640, 640), :])


_TC_PARAMS = pltpu.CompilerParams(vmem_limit_bytes=100 * 1024 * 1024)


def _tc0_body(batch_ref, counts_ref):
    bm = batch_ref[...]
    for g in range(G):
        counts_ref[g] = jnp.sum((bm == g).astype(I32))


def _counts_from_batch(batch_m):
    return pl.pallas_call(
        _tc0_body,
        out_shape=jax.ShapeDtypeStruct((G,), I32),
        in_specs=[pl.BlockSpec(memory_space=pltpu.VMEM)],
        out_specs=pl.BlockSpec(memory_space=pltpu.SMEM),
        compiler_params=_TC_PARAMS,
    )(batch_m)


def _tcpre_body(h_ref, w_ref, deg_ref, y_ref, dinv_ref):
    xw = jnp.dot(h_ref[...], w_ref[...], preferred_element_type=F32)
    deg = deg_ref[0] + deg_ref[1] + 1.0
    dinv = lax.rsqrt(deg)
    dinv_ref[...] = dinv
    y3 = xw.reshape(RROWS, 128, H) * dinv[:, :, None]
    y_ref[...] = y3.reshape(NP, H)


def _tc_pre(h, Wc, deg3):
    return pl.pallas_call(
        _tcpre_body,
        out_shape=(jax.ShapeDtypeStruct((NP, H), F32),
                   jax.ShapeDtypeStruct((RROWS, 128), F32)),
        in_specs=[pl.BlockSpec(memory_space=pltpu.VMEM)] * 3,
        out_specs=(pl.BlockSpec(memory_space=pltpu.VMEM),) * 2,
        compiler_params=_TC_PARAMS,
    )(h, Wc, deg3)


def _tc3_body(agg0_ref, agg1_ref, y_ref, dinv_ref, bc_ref, gm_ref, bt_ref,
              p_ref, alive_ref, batch_ref, counts_ref, pn_ref, flat_ref,
              h_ref, kept_ref, cnew_ref, fout_ref, s_scr, bb_scr):
    dinv = dinv_ref[...]
    conv = dinv[:, :, None] * (agg0_ref[...] + agg1_ref[...] + y_ref[...])
    conv = conv + bc_ref[...].reshape(1, 1, H)
    alive = alive_ref[...]
    alive3 = alive[:, :, None]
    cnt = counts_ref[0]
    for g in range(1, G):
        cnt = cnt + counts_ref[g]
    cntf = jnp.maximum(cnt.astype(F32), 1.0)
    mean = jnp.sum(jnp.sum(conv * alive3, axis=0), axis=0) / cntf
    cen = conv - mean[None, None, :]
    var = jnp.sum(jnp.sum(cen * cen * alive3, axis=0), axis=0) / cntf
    gm = gm_ref[...].reshape(1, 1, H)
    bt = bt_ref[...].reshape(1, 1, H)
    h2 = jax.nn.relu(gm * cen / jnp.sqrt(var + EPS)[None, None, :] + bt)

    p2 = p_ref[...]
    zrows = []
    for r in range(RROWS):
        zrows.append(lax.dot_general(p2, h2[r], (((1,), (1,)), ((), ())),
                                     preferred_element_type=F32))
    z = jnp.concatenate(zrows, axis=0)
    s = jnp.tanh(z / pn_ref[0, 0])
    s_scr[...] = s

    bm = batch_ref[...]
    bb = jnp.where(alive > 0.0, bm, -1)
    bb_scr[...] = bb
    iidx = (lax.broadcasted_iota(I32, (RROWS, 128), 0) * 128
            + lax.broadcasted_iota(I32, (RROWS, 128), 1))
    s3 = s[:, :, None]
    i3 = iidx[:, :, None]
    b3 = bb[:, :, None]

    def jbody(jt, rank):
        sj = s_scr[pl.ds(jt, 1), :].reshape(1, 1, 128)
        bj = bb_scr[pl.ds(jt, 1), :].reshape(1, 1, 128)
        ij = (jt * 128
              + lax.broadcasted_iota(I32, (1, 1, 128), 2))
        above = (sj > s3) | ((sj == s3) & (ij < i3))
        return rank + jnp.sum((above & (bj == b3)).astype(I32), axis=2)

    rank = lax.fori_loop(0, RROWS, jbody, jnp.zeros((RROWS, 128), I32))

    kvals = []
    for g in range(G):
        kg = (4 * counts_ref[g] + 4) // 5
        kvals.append(kg)
        cnew_ref[g] = kg
    kb = jnp.zeros((RROWS, 128), I32)
    for g in range(G):
        kb = jnp.where(bm == g, kvals[g], kb)
    kept = (alive > 0.0) & (rank < kb)
    keptf = kept.astype(F32)
    kept_ref[...] = keptf
    h_new = h2 * s3 * keptf[:, :, None]
    h_ref[...] = h_new.reshape(NP, H)

    means = []
    maxs = []
    ninf = jnp.float32(-jnp.inf)
    for g in range(G):
        mf3 = (kept & (bm == g)).astype(F32)[:, :, None]
        sum_g = jnp.sum(jnp.sum(h_new * mf3, axis=0), axis=0)
        kf = jnp.maximum(kvals[g].astype(F32), 1.0)
        means.append(sum_g / kf)
        maxs.append(jnp.max(jnp.max(jnp.where(mf3 > 0.5, h_new, ninf), axis=0),
                            axis=0))
    flat = jnp.concatenate([jnp.stack(means), jnp.stack(maxs)], axis=1)
    fout_ref[...] = flat_ref[...] + flat


def _tc_post(agg0, agg1, y3, dinv_m, bc, gm, bt, p, alive_m, batch_m, counts,
             pn, flat):
    return pl.pallas_call(
        _tc3_body,
        out_shape=(jax.ShapeDtypeStruct((NP, H), F32),
                   jax.ShapeDtypeStruct((RROWS, 128), F32),
                   jax.ShapeDtypeStruct((G,), I32),
                   jax.ShapeDtypeStruct((G, 2 * H), F32)),
        in_specs=([pl.BlockSpec(memory_space=pltpu.VMEM)] * 10
                  + [pl.BlockSpec(memory_space=pltpu.SMEM),
                     pl.BlockSpec(memory_space=pltpu.SMEM),
                     pl.BlockSpec(memory_space=pltpu.VMEM)]),
        out_specs=(pl.BlockSpec(memory_space=pltpu.VMEM),
                   pl.BlockSpec(memory_space=pltpu.VMEM),
                   pl.BlockSpec(memory_space=pltpu.SMEM),
                   pl.BlockSpec(memory_space=pltpu.VMEM)),
        scratch_shapes=[pltpu.VMEM((RROWS, 128), F32),
                        pltpu.VMEM((RROWS, 128), I32)],
        compiler_params=_TC_PARAMS,
    )(agg0, agg1, y3, dinv_m, bc, gm, bt, p, alive_m, batch_m, counts, pn,
      flat)


def _mlp_body(f_ref, w1_ref, b1_ref, w2_ref, b2_ref, o_ref):
    hid = jax.nn.relu(
        jnp.dot(f_ref[...], w1_ref[...], preferred_element_type=F32)
        + b1_ref[...])
    o_ref[...] = (jnp.dot(hid, w2_ref[...], preferred_element_type=F32)
                  + b2_ref[...])


def _mlp(flat, W1, b1, W2, b2):
    return pl.pallas_call(
        _mlp_body,
        out_shape=jax.ShapeDtypeStruct((G, W2.shape[1]), F32),
        in_specs=[pl.BlockSpec(memory_space=pltpu.VMEM)] * 5,
        out_specs=pl.BlockSpec(memory_space=pltpu.VMEM),
        compiler_params=_TC_PARAMS,
    )(flat, W1, b1[None, :], W2, b2[None, :])


def kernel(inputs, edge_index, batch, edge_weight, Wc0, bc0, gamma0, beta0,
           p0, Wc1, bc1, gamma1, beta1, p1, Wc2, bc2, gamma2, beta2, p2,
           W1, b1, W2, b2):
    row = edge_index[0].astype(I32)
    col = edge_index[1].astype(I32)
    zpad = jnp.zeros((EP - E,), I32)
    row_p = jnp.concatenate([row, zpad])
    col_p = jnp.concatenate([col, zpad])
    ew_p = jnp.concatenate([edge_weight.astype(F32),
                            jnp.zeros((EP - E,), F32)])
    batch_m = jnp.concatenate([batch.astype(I32),
                               jnp.full((NP - N,), G, I32)]).reshape(RROWS, 128)
    h = jnp.concatenate([inputs.astype(F32), jnp.zeros((NP - N, H), F32)],
                        axis=0)
    alive_m = (jnp.arange(NP) < N).astype(F32).reshape(RROWS, 128)

    counts = _counts_from_batch(batch_m)
    flat = jnp.zeros((G, 2 * H), F32)
    ew_cur = ew_p
    blocks = [(Wc0, bc0, gamma0, beta0, p0), (Wc1, bc1, gamma1, beta1, p1),
              (Wc2, bc2, gamma2, beta2, p2)]
    for (Wc, bc, gm, bt, p) in blocks:
        ewn, degflat = _sc_deg(row_p, col_p, ew_cur, alive_m.reshape(NP))
        y, dinv_m = _tc_pre(h, Wc, degflat.reshape(2, RROWS, 128))
        aggpart = _sc_agg(y, row_p, col_p, ewn)
        pn = jnp.linalg.norm(p).reshape(1, 1)
        h, alive_m, counts, flat = _tc_post(
            aggpart[0].reshape(RROWS, 128, H), aggpart[1].reshape(RROWS, 128, H),
            y.reshape(RROWS, 128, H), dinv_m, bc.reshape(1, H),
            gm.reshape(1, H), bt.reshape(1, H), p.reshape(1, H),
            alive_m, batch_m, counts, pn, flat)
        ew_cur = ewn
    return _mlp(flat, W1, b1, W2, b2)
